# baseline (speedup 1.0000x reference)
import functools

import jax
import jax.numpy as jnp
from jax import lax
from jax.experimental import pallas as pl
from jax.experimental.pallas import tpu as pltpu
from jax.experimental.pallas import tpu_sc as plsc

WIDTH_FULL = 17
WIDTH_PLAYABLE = 15
HEIGHT = 11
TYPE_EMB_DIM = 8
K = 16384
OUT_D = 14
LANES = 16
NC, NS = 2, 16
NW = NC * NS
IPW = K // NW
GROUPS = IPW // LANES
N_EMB_WORDS = 4 * TYPE_EMB_DIM


def _sc_body(emb_hbm, tid_hbm, h1_hbm, h2_hbm, out_hbm,
             emb2_v, emb_v, in_v, col_v, sem_in):
    wid = lax.axis_index("s") * NC + lax.axis_index("c")
    base = wid * IPW

    cps = [
        pltpu.async_copy(emb_hbm, emb2_v, sem_in),
        pltpu.async_copy(tid_hbm.at[pl.ds(base, IPW)],
                         in_v.at[pl.ds(0, IPW)], sem_in),
        pltpu.async_copy(h1_hbm.at[pl.ds(base, IPW)],
                         in_v.at[pl.ds(IPW, IPW)], sem_in),
        pltpu.async_copy(h2_hbm.at[pl.ds(base, IPW)],
                         in_v.at[pl.ds(2 * IPW, IPW)], sem_in),
    ]
    for cp in cps:
        cp.wait()

    ones = jnp.full((LANES,), 1.0, dtype=jnp.float32)

    lane = lax.iota(jnp.int32, LANES)
    lrow = lane // TYPE_EMB_DIM
    lcol = lane - lrow * TYPE_EMB_DIM
    for k in range(2):
        emb_v[pl.ds(k * LANES, LANES)] = plsc.load_gather(
            emb2_v, [lrow + 2 * k, lcol])

    def group(g, carry):
        off = g * LANES
        tid = in_v[pl.ds(off, LANES)]

        tid8 = tid * TYPE_EMB_DIM

        def emb_col(j, carry2):
            col_v[j, pl.ds(off, LANES)] = plsc.load_gather(emb_v, [tid8 + j])
            return carry2

        lax.fori_loop(0, TYPE_EMB_DIM, emb_col, 0)

        def hex_cols(k, carry2):
            h = in_v[pl.ds((k + 1) * IPW + off, LANES)]
            jb = TYPE_EMB_DIM + 3 * k
            y = lax.div(h, WIDTH_FULL)
            x = h - y * WIDTH_FULL
            xf = jnp.minimum(x, WIDTH_PLAYABLE - 1).astype(jnp.float32)
            col_v[jb, pl.ds(off, LANES)] = xf * (1.0 / (WIDTH_PLAYABLE - 1))
            col_v[jb + 1, pl.ds(off, LANES)] = (
                y.astype(jnp.float32) * (1.0 / (HEIGHT - 1)))
            col_v[jb + 2, pl.ds(off, LANES)] = ones
            return carry2

        lax.fori_loop(0, 2, hex_cols, 0)
        return carry

    lax.fori_loop(0, GROUPS, group, 0)

    pltpu.sync_copy(col_v, out_hbm.at[:, pl.ds(base, IPW)])


@functools.cache
def _build():
    mesh = plsc.VectorSubcoreMesh(
        core_axis_name="c", subcore_axis_name="s",
        num_cores=NC, num_subcores=NS)
    return pl.kernel(
        _sc_body,
        out_type=jax.ShapeDtypeStruct((OUT_D, K), jnp.float32),
        mesh=mesh,
        compiler_params=pltpu.CompilerParams(
            needs_layout_passes=False, use_tc_tiling_on_sc=True),
        scratch_types=[
            pltpu.VMEM((4, TYPE_EMB_DIM), jnp.float32),
            pltpu.VMEM((N_EMB_WORDS,), jnp.float32),
            pltpu.VMEM((3 * IPW,), jnp.int32),
            pltpu.VMEM((OUT_D, IPW), jnp.float32),
            pltpu.SemaphoreType.DMA,
        ],
    )


def kernel(type_emb, type_ids, hex1, hex2):
    emb = type_emb.astype(jnp.float32)
    tid = type_ids.astype(jnp.int32)
    h1 = hex1.astype(jnp.int32)
    h2 = hex2.astype(jnp.int32)
    return _build()(emb, tid, h1, h2).T


# speedup vs baseline: 1.0632x; 1.0632x over previous
import functools

import jax
import jax.numpy as jnp
from jax import lax
from jax.experimental import pallas as pl
from jax.experimental.pallas import tpu as pltpu
from jax.experimental.pallas import tpu_sc as plsc

WIDTH_FULL = 17
WIDTH_PLAYABLE = 15
HEIGHT = 11
TYPE_EMB_DIM = 8
K = 16384
OUT_D = 14
LANES = 16
NC, NS = 2, 16
NW = NC * NS
IPW = K // NW
GROUPS = IPW // LANES
N_EMB_WORDS = 4 * TYPE_EMB_DIM


def _sc_body(emb_hbm, tid_hbm, h1_hbm, h2_hbm, out_hbm,
             emb_v, tid_v, h1_v, h2_v, col_v, sem_in):
    wid = lax.axis_index("s") * NC + lax.axis_index("c")
    base = wid * IPW

    cps = [
        pltpu.async_copy(emb_hbm, emb_v, sem_in),
        pltpu.async_copy(tid_hbm.at[pl.ds(base, IPW)], tid_v, sem_in),
        pltpu.async_copy(h1_hbm.at[pl.ds(base, IPW)], h1_v, sem_in),
        pltpu.async_copy(h2_hbm.at[pl.ds(base, IPW)], h2_v, sem_in),
    ]
    for cp in cps:
        cp.wait()

    @plsc.parallel_loop(0, GROUPS)
    def group(g):
        off = g * LANES
        tid = tid_v[pl.ds(off, LANES)]
        h1 = h1_v[pl.ds(off, LANES)]
        h2 = h2_v[pl.ds(off, LANES)]

        tid8 = tid * TYPE_EMB_DIM
        for j in range(TYPE_EMB_DIM):
            col_v[j, pl.ds(off, LANES)] = plsc.load_gather(emb_v, [tid8 + j])

        for h, jb in ((h1, TYPE_EMB_DIM), (h2, TYPE_EMB_DIM + 3)):
            y = lax.div(h, WIDTH_FULL)
            x = h - y * WIDTH_FULL
            valid = h >= 0
            xf = jnp.minimum(x, WIDTH_PLAYABLE - 1).astype(jnp.float32)
            yf = jnp.minimum(y, HEIGHT - 1).astype(jnp.float32)
            col_v[jb, pl.ds(off, LANES)] = jnp.where(
                valid, xf * (1.0 / (WIDTH_PLAYABLE - 1)), 0.0)
            col_v[jb + 1, pl.ds(off, LANES)] = jnp.where(
                valid, yf * (1.0 / (HEIGHT - 1)), 0.0)
            col_v[jb + 2, pl.ds(off, LANES)] = jnp.where(valid, 1.0, 0.0)

    pltpu.sync_copy(col_v, out_hbm.at[:, pl.ds(base, IPW)])


@functools.cache
def _build():
    mesh = plsc.VectorSubcoreMesh(
        core_axis_name="c", subcore_axis_name="s",
        num_cores=NC, num_subcores=NS)
    return pl.kernel(
        _sc_body,
        out_type=jax.ShapeDtypeStruct((OUT_D, K), jnp.float32),
        mesh=mesh,
        compiler_params=pltpu.CompilerParams(
            needs_layout_passes=False, use_tc_tiling_on_sc=True),
        scratch_types=[
            pltpu.VMEM((N_EMB_WORDS,), jnp.float32),
            pltpu.VMEM((IPW,), jnp.int32),
            pltpu.VMEM((IPW,), jnp.int32),
            pltpu.VMEM((IPW,), jnp.int32),
            pltpu.VMEM((OUT_D, IPW), jnp.float32),
            pltpu.SemaphoreType.DMA,
        ],
    )


def kernel(type_emb, type_ids, hex1, hex2):
    emb = type_emb.reshape(-1).astype(jnp.float32)
    tid = type_ids.astype(jnp.int32)
    h1 = hex1.astype(jnp.int32)
    h2 = hex2.astype(jnp.int32)
    return _build()(emb, tid, h1, h2).T
